# baseline (speedup 1.0000x reference)
import functools

import jax
import jax.numpy as jnp
from jax.experimental import pallas as pl
from jax.experimental.pallas import tpu as pltpu

D = 64
K = 1024
BETA = 0.25
N_ROWS = 16 * 576
BLOCK_ROWS = 768
N_BLOCKS = N_ROWS // BLOCK_ROWS


def _vq_body(flat_ref, emb_ref, q_ref, idx_ref, loss_ref, perp_ref,
             counts_ref, sqerr_ref):
    step = pl.program_id(0)

    x = flat_ref[...]
    e = emb_ref[...]

    sx = jnp.sum(x * x, axis=1, keepdims=True)
    se = jnp.sum(e * e, axis=1)
    dots = jax.lax.dot_general(
        x, e, dimension_numbers=(((1,), (1,)), ((), ())),
        preferred_element_type=jnp.float32)
    dist = (sx + se[None, :]) - 2.0 * dots

    dmin = jnp.min(dist, axis=1, keepdims=True)
    iota_k = jax.lax.broadcasted_iota(jnp.int32, (1, K), 1)
    cand = jnp.where(dist == dmin, iota_k, K)
    idx = jnp.min(cand, axis=1).astype(jnp.int32)
    idx_ref[0, 0, :] = idx

    onehot = (cand == idx[:, None]).astype(jnp.float32)
    q = jax.lax.dot_general(
        onehot, e, dimension_numbers=(((1,), (0,)), ((), ())),
        preferred_element_type=jnp.float32,
        precision=jax.lax.Precision.HIGHEST)
    q_ref[...] = q

    counts_part = jnp.sum(onehot, axis=0, keepdims=True)
    diff = q - x
    sq_part = jnp.sum(diff * diff)

    @pl.when(step == 0)
    def _init():
        counts_ref[...] = counts_part
        sqerr_ref[0, 0] = sq_part

    @pl.when(step != 0)
    def _acc():
        counts_ref[...] += counts_part
        sqerr_ref[0, 0] += sq_part

    @pl.when(step == N_BLOCKS - 1)
    def _final():
        loss_ref[0, 0] = BETA * sqerr_ref[0, 0] / float(N_ROWS * D)
        avg = counts_ref[...] / float(N_ROWS)
        ent = jnp.sum(avg * jnp.log(avg + 1e-10))
        perp_ref[0, 0] = jnp.exp(-ent)


@functools.partial(jax.jit)
def kernel(z, embedding):
    flat = z.reshape(N_ROWS, D)
    q, idx3, loss, perp = pl.pallas_call(
        _vq_body,
        grid=(N_BLOCKS,),
        in_specs=[
            pl.BlockSpec((BLOCK_ROWS, D), lambda i: (i, 0)),
            pl.BlockSpec((K, D), lambda i: (0, 0)),
        ],
        out_specs=[
            pl.BlockSpec((BLOCK_ROWS, D), lambda i: (i, 0)),
            pl.BlockSpec((1, 1, BLOCK_ROWS), lambda i: (i, 0, 0)),
            pl.BlockSpec(memory_space=pltpu.SMEM),
            pl.BlockSpec(memory_space=pltpu.SMEM),
        ],
        out_shape=[
            jax.ShapeDtypeStruct((N_ROWS, D), jnp.float32),
            jax.ShapeDtypeStruct((N_BLOCKS, 1, BLOCK_ROWS), jnp.int32),
            jax.ShapeDtypeStruct((1, 1), jnp.float32),
            jax.ShapeDtypeStruct((1, 1), jnp.float32),
        ],
        scratch_shapes=[
            pltpu.VMEM((1, K), jnp.float32),
            pltpu.SMEM((1, 1), jnp.float32),
        ],
    )(flat, embedding)
    quantized_st = q.reshape(z.shape)
    encoding_indices = idx3.reshape(N_ROWS)
    return quantized_st, loss[0, 0], perp[0, 0], encoding_indices


# speedup vs baseline: 1.0857x; 1.0857x over previous
import functools

import jax
import jax.numpy as jnp
from jax.experimental import pallas as pl
from jax.experimental.pallas import tpu as pltpu

D = 64
K = 1024
BETA = 0.25
N_ROWS = 16 * 576
BLOCK_ROWS = 768
N_BLOCKS = N_ROWS // BLOCK_ROWS


def _vq_body(flat_ref, emb_ref, q_ref, idx_ref, loss_ref, perp_ref,
             counts_ref, sqerr_ref):
    step = pl.program_id(0)

    x = flat_ref[...]
    e = emb_ref[...]

    sx = jnp.sum(x * x, axis=1, keepdims=True)
    se = jnp.sum(e * e, axis=1)
    dots = jax.lax.dot_general(
        x, e, dimension_numbers=(((1,), (1,)), ((), ())),
        preferred_element_type=jnp.float32)
    dist = (sx + se[None, :]) - 2.0 * dots

    idx = jnp.argmin(dist, axis=1).astype(jnp.int32)
    idx_ref[0, 0, :] = idx

    onehot = (idx[:, None] == jax.lax.broadcasted_iota(jnp.int32, (1, K), 1)
              ).astype(jnp.float32)
    q = jax.lax.dot_general(
        onehot, e, dimension_numbers=(((1,), (0,)), ((), ())),
        preferred_element_type=jnp.float32,
        precision=jax.lax.Precision.HIGHEST)
    q_ref[...] = q

    counts_part = jnp.sum(onehot, axis=0, keepdims=True)
    diff = q - x
    sq_part = jnp.sum(diff * diff)

    @pl.when(step == 0)
    def _init():
        counts_ref[...] = counts_part
        sqerr_ref[0, 0] = sq_part

    @pl.when(step != 0)
    def _acc():
        counts_ref[...] += counts_part
        sqerr_ref[0, 0] += sq_part

    @pl.when(step == N_BLOCKS - 1)
    def _final():
        loss_ref[0, 0] = BETA * sqerr_ref[0, 0] / float(N_ROWS * D)
        avg = counts_ref[...] / float(N_ROWS)
        ent = jnp.sum(avg * jnp.log(avg + 1e-10))
        perp_ref[0, 0] = jnp.exp(-ent)


@functools.partial(jax.jit)
def kernel(z, embedding):
    flat = z.reshape(N_ROWS, D)
    q, idx3, loss, perp = pl.pallas_call(
        _vq_body,
        grid=(N_BLOCKS,),
        in_specs=[
            pl.BlockSpec((BLOCK_ROWS, D), lambda i: (i, 0)),
            pl.BlockSpec((K, D), lambda i: (0, 0)),
        ],
        out_specs=[
            pl.BlockSpec((BLOCK_ROWS, D), lambda i: (i, 0)),
            pl.BlockSpec((1, 1, BLOCK_ROWS), lambda i: (i, 0, 0)),
            pl.BlockSpec(memory_space=pltpu.SMEM),
            pl.BlockSpec(memory_space=pltpu.SMEM),
        ],
        out_shape=[
            jax.ShapeDtypeStruct((N_ROWS, D), jnp.float32),
            jax.ShapeDtypeStruct((N_BLOCKS, 1, BLOCK_ROWS), jnp.int32),
            jax.ShapeDtypeStruct((1, 1), jnp.float32),
            jax.ShapeDtypeStruct((1, 1), jnp.float32),
        ],
        scratch_shapes=[
            pltpu.VMEM((1, K), jnp.float32),
            pltpu.SMEM((1, 1), jnp.float32),
        ],
    )(flat, embedding)
    quantized_st = q.reshape(z.shape)
    encoding_indices = idx3.reshape(N_ROWS)
    return quantized_st, loss[0, 0], perp[0, 0], encoding_indices


# speedup vs baseline: 1.1382x; 1.0484x over previous
import functools

import jax
import jax.numpy as jnp
from jax.experimental import pallas as pl
from jax.experimental.pallas import tpu as pltpu

D = 64
K = 1024
BETA = 0.25
N_ROWS = 16 * 576
BLOCK_ROWS = 2304
N_BLOCKS = N_ROWS // BLOCK_ROWS


def _vq_body(flat_ref, emb_ref, q_ref, idx_ref, loss_ref, perp_ref,
             counts_ref, sqerr_ref):
    step = pl.program_id(0)

    x = flat_ref[...]
    e = emb_ref[...]

    sx = jnp.sum(x * x, axis=1, keepdims=True)
    se = jnp.sum(e * e, axis=1)
    dots = jax.lax.dot_general(
        x, e, dimension_numbers=(((1,), (1,)), ((), ())),
        preferred_element_type=jnp.float32)
    dist = (sx + se[None, :]) - 2.0 * dots

    idx = jnp.argmin(dist, axis=1).astype(jnp.int32)
    idx_ref[0, 0, :] = idx

    onehot = (idx[:, None] == jax.lax.broadcasted_iota(jnp.int32, (1, K), 1)
              ).astype(jnp.float32)
    q = jax.lax.dot_general(
        onehot, e, dimension_numbers=(((1,), (0,)), ((), ())),
        preferred_element_type=jnp.float32,
        precision=jax.lax.Precision.HIGHEST)
    q_ref[...] = q

    counts_part = jnp.sum(onehot, axis=0, keepdims=True)
    diff = q - x
    sq_part = jnp.sum(diff * diff)

    @pl.when(step == 0)
    def _init():
        counts_ref[...] = counts_part
        sqerr_ref[0, 0] = sq_part

    @pl.when(step != 0)
    def _acc():
        counts_ref[...] += counts_part
        sqerr_ref[0, 0] += sq_part

    @pl.when(step == N_BLOCKS - 1)
    def _final():
        loss_ref[0, 0] = BETA * sqerr_ref[0, 0] / float(N_ROWS * D)
        avg = counts_ref[...] / float(N_ROWS)
        ent = jnp.sum(avg * jnp.log(avg + 1e-10))
        perp_ref[0, 0] = jnp.exp(-ent)


@functools.partial(jax.jit)
def kernel(z, embedding):
    flat = z.reshape(N_ROWS, D)
    q, idx3, loss, perp = pl.pallas_call(
        _vq_body,
        grid=(N_BLOCKS,),
        in_specs=[
            pl.BlockSpec((BLOCK_ROWS, D), lambda i: (i, 0)),
            pl.BlockSpec((K, D), lambda i: (0, 0)),
        ],
        out_specs=[
            pl.BlockSpec((BLOCK_ROWS, D), lambda i: (i, 0)),
            pl.BlockSpec((1, 1, BLOCK_ROWS), lambda i: (i, 0, 0)),
            pl.BlockSpec(memory_space=pltpu.SMEM),
            pl.BlockSpec(memory_space=pltpu.SMEM),
        ],
        out_shape=[
            jax.ShapeDtypeStruct((N_ROWS, D), jnp.float32),
            jax.ShapeDtypeStruct((N_BLOCKS, 1, BLOCK_ROWS), jnp.int32),
            jax.ShapeDtypeStruct((1, 1), jnp.float32),
            jax.ShapeDtypeStruct((1, 1), jnp.float32),
        ],
        scratch_shapes=[
            pltpu.VMEM((1, K), jnp.float32),
            pltpu.SMEM((1, 1), jnp.float32),
        ],
    )(flat, embedding)
    quantized_st = q.reshape(z.shape)
    encoding_indices = idx3.reshape(N_ROWS)
    return quantized_st, loss[0, 0], perp[0, 0], encoding_indices


# speedup vs baseline: 1.7886x; 1.5714x over previous
import functools

import jax
import jax.numpy as jnp
from jax.experimental import pallas as pl
from jax.experimental.pallas import tpu as pltpu

D = 64
K = 1024
BETA = 0.25
N_ROWS = 16 * 576
BLOCK_ROWS = 2304
N_BLOCKS = N_ROWS // BLOCK_ROWS


def _vq_body(flat_ref, emb_ref, q_ref, idx_ref, loss_ref, perp_ref,
             counts_ref, sqerr_ref):
    step = pl.program_id(0)

    x = flat_ref[...]
    e = emb_ref[...]

    sx = jnp.sum(x * x, axis=1, keepdims=True)
    se = jnp.sum(e * e, axis=1)
    dots = jax.lax.dot_general(
        x, e, dimension_numbers=(((1,), (1,)), ((), ())),
        preferred_element_type=jnp.float32)
    dist = (sx + se[None, :]) - 2.0 * dots

    idx = jnp.argmin(dist, axis=1).astype(jnp.int32)
    idx_ref[0, 0, :] = idx

    onehot = (idx[:, None] == jax.lax.broadcasted_iota(jnp.int32, (1, K), 1)
              ).astype(jnp.float32)
    q = jax.lax.dot_general(
        onehot, e, dimension_numbers=(((1,), (0,)), ((), ())),
        preferred_element_type=jnp.float32)
    q_ref[...] = q

    counts_part = jax.lax.dot_general(
        jnp.ones((1, BLOCK_ROWS), jnp.float32), onehot,
        dimension_numbers=(((1,), (0,)), ((), ())),
        preferred_element_type=jnp.float32)
    diff = q - x
    sq_part = jnp.sum(diff * diff)

    @pl.when(step == 0)
    def _init():
        counts_ref[...] = counts_part
        sqerr_ref[0, 0] = sq_part

    @pl.when(step != 0)
    def _acc():
        counts_ref[...] += counts_part
        sqerr_ref[0, 0] += sq_part

    @pl.when(step == N_BLOCKS - 1)
    def _final():
        loss_ref[0, 0] = BETA * sqerr_ref[0, 0] / float(N_ROWS * D)
        avg = counts_ref[...] / float(N_ROWS)
        ent = jnp.sum(avg * jnp.log(avg + 1e-10))
        perp_ref[0, 0] = jnp.exp(-ent)


@functools.partial(jax.jit)
def kernel(z, embedding):
    flat = z.reshape(N_ROWS, D)
    q, idx3, loss, perp = pl.pallas_call(
        _vq_body,
        grid=(N_BLOCKS,),
        in_specs=[
            pl.BlockSpec((BLOCK_ROWS, D), lambda i: (i, 0)),
            pl.BlockSpec((K, D), lambda i: (0, 0)),
        ],
        out_specs=[
            pl.BlockSpec((BLOCK_ROWS, D), lambda i: (i, 0)),
            pl.BlockSpec((1, 1, BLOCK_ROWS), lambda i: (i, 0, 0)),
            pl.BlockSpec(memory_space=pltpu.SMEM),
            pl.BlockSpec(memory_space=pltpu.SMEM),
        ],
        out_shape=[
            jax.ShapeDtypeStruct((N_ROWS, D), jnp.float32),
            jax.ShapeDtypeStruct((N_BLOCKS, 1, BLOCK_ROWS), jnp.int32),
            jax.ShapeDtypeStruct((1, 1), jnp.float32),
            jax.ShapeDtypeStruct((1, 1), jnp.float32),
        ],
        scratch_shapes=[
            pltpu.VMEM((1, K), jnp.float32),
            pltpu.SMEM((1, 1), jnp.float32),
        ],
    )(flat, embedding)
    quantized_st = q.reshape(z.shape)
    encoding_indices = idx3.reshape(N_ROWS)
    return quantized_st, loss[0, 0], perp[0, 0], encoding_indices
